# baseline (speedup 1.0000x reference)
import functools

import jax
import jax.numpy as jnp
from jax import lax
from jax.experimental import pallas as pl
from jax.experimental.pallas import tpu as pltpu
from jax.experimental.pallas import tpu_sc as plsc

N = 10000
E = 320000
D_IN = 128
D_H = 256
D_OUT = 128
B = 128
HW = D_H // 2

NC = 2
NS = 16
NP = 10112
RPT = NP // NS
CHUNK = 64
CH = 320
CH2 = 160
IBF = 64
IBE = 40
EP = NS * CH * CHUNK

_BLK = 400
_NBLK = N // _BLK


def _sc_mesh():
    return plsc.VectorSubcoreMesh(core_axis_name="c", subcore_axis_name="s",
                                  num_cores=NC)


def _edge_loop(y_hbm, src_hbm, dst_hbm, acc, src_v, dst_v, rows_v, sem,
               i_src, i_dst, nblk, ib):
    @pl.loop(0, nblk)
    def _(b):
        base = b * ib
        pltpu.sync_copy(src_hbm.at[i_src, pl.ds(base, ib)], src_v)
        pltpu.sync_copy(dst_hbm.at[i_dst, pl.ds(base, ib)], dst_v)

        @pl.loop(0, ib)
        def _(j):
            pltpu.async_copy(y_hbm.at[src_v.at[j]], rows_v, sem).wait()
            pltpu.sync_copy(rows_v, acc.at[dst_v.at[j]], add=True)


@functools.partial(
    pl.kernel,
    out_type=[jax.ShapeDtypeStruct((NP, 16), jnp.float32),
              jax.ShapeDtypeStruct((NP, 16), jnp.float32)],
    mesh=_sc_mesh(),
    compiler_params=pltpu.CompilerParams(use_tc_tiling_on_sc=False),
    scratch_types=[
        pltpu.VMEM_SHARED((NP, 16), jnp.float32),
        pltpu.VMEM((CH2, CHUNK), jnp.int32),
        pltpu.VMEM((CHUNK, 16), jnp.float32),
    ],
)
def _deg_kernel(dst_hbm, ones_hbm, zeros_hbm, outa_hbm, outb_hbm,
                acc, dst_v, ones_v):
    c = lax.axis_index("c")
    s = lax.axis_index("s")
    wid = c * NS + s
    r0 = s * RPT
    pltpu.sync_copy(dst_hbm.at[wid], dst_v)
    pltpu.sync_copy(ones_hbm, ones_v)
    pltpu.sync_copy(zeros_hbm.at[pl.ds(r0, RPT)], acc.at[pl.ds(r0, RPT)])
    plsc.subcore_barrier()

    @pl.loop(0, CH2)
    def _(j):
        pltpu.sync_copy(ones_v, acc.at[dst_v.at[j]], add=True)

    plsc.subcore_barrier()

    @pl.when(c == 0)
    def _():
        pltpu.sync_copy(acc.at[pl.ds(r0, RPT)], outa_hbm.at[pl.ds(r0, RPT)])

    @pl.when(c == 1)
    def _():
        pltpu.sync_copy(acc.at[pl.ds(r0, RPT)], outb_hbm.at[pl.ds(r0, RPT)])


@functools.partial(
    pl.kernel,
    out_type=[jax.ShapeDtypeStruct((NP, D_IN), jnp.float32),
              jax.ShapeDtypeStruct((NP, D_IN), jnp.float32)],
    mesh=_sc_mesh(),
    compiler_params=pltpu.CompilerParams(use_tc_tiling_on_sc=False),
    scratch_types=[
        pltpu.VMEM_SHARED((NP, D_IN), jnp.float32),
        pltpu.VMEM((IBE, CHUNK), jnp.int32),
        pltpu.VMEM((IBE, CHUNK), jnp.int32),
        pltpu.VMEM((CHUNK, D_IN), jnp.float32),
        pltpu.SemaphoreType.DMA,
    ],
)
def _scatter_es(y_hbm, src_hbm, dst_hbm, zeros_hbm, outa_hbm, outb_hbm,
                acc, src_v, dst_v, rows_v, sem):
    c = lax.axis_index("c")
    s = lax.axis_index("s")
    wid = c * NS + s
    r0 = s * RPT

    @pl.when(c == 0)
    def _():
        pltpu.sync_copy(y_hbm.at[pl.ds(r0, RPT)], acc.at[pl.ds(r0, RPT)])

    @pl.when(c == 1)
    def _():
        pltpu.sync_copy(zeros_hbm.at[pl.ds(r0, RPT)], acc.at[pl.ds(r0, RPT)])

    plsc.subcore_barrier()
    _edge_loop(y_hbm, src_hbm, dst_hbm, acc, src_v, dst_v, rows_v, sem,
               wid, wid, CH2 // IBE, IBE)
    plsc.subcore_barrier()

    @pl.when(c == 0)
    def _():
        pltpu.sync_copy(acc.at[pl.ds(r0, RPT)], outa_hbm.at[pl.ds(r0, RPT)])

    @pl.when(c == 1)
    def _():
        pltpu.sync_copy(acc.at[pl.ds(r0, RPT)], outb_hbm.at[pl.ds(r0, RPT)])


@functools.partial(
    pl.kernel,
    out_type=[jax.ShapeDtypeStruct((NP, HW), jnp.float32),
              jax.ShapeDtypeStruct((NP, HW), jnp.float32)],
    mesh=_sc_mesh(),
    compiler_params=pltpu.CompilerParams(use_tc_tiling_on_sc=False),
    scratch_types=[
        pltpu.VMEM_SHARED((NP, HW), jnp.float32),
        pltpu.VMEM((IBF, CHUNK), jnp.int32),
        pltpu.VMEM((IBF, CHUNK), jnp.int32),
        pltpu.VMEM((CHUNK, HW), jnp.float32),
        pltpu.SemaphoreType.DMA,
    ],
)
def _scatter_fs(yl_hbm, yr_hbm, src_hbm, dst_hbm, outl_hbm, outr_hbm,
                acc, src_v, dst_v, rows_v, sem):
    c = lax.axis_index("c")
    s = lax.axis_index("s")
    r0 = s * RPT

    @pl.when(c == 0)
    def _():
        pltpu.sync_copy(yl_hbm.at[pl.ds(r0, RPT)], acc.at[pl.ds(r0, RPT)])

    @pl.when(c == 1)
    def _():
        pltpu.sync_copy(yr_hbm.at[pl.ds(r0, RPT)], acc.at[pl.ds(r0, RPT)])

    plsc.subcore_barrier()

    @pl.when(c == 0)
    def _():
        _edge_loop(yl_hbm, src_hbm, dst_hbm, acc, src_v, dst_v, rows_v, sem,
                   s, s, CH // IBF, IBF)

    @pl.when(c == 1)
    def _():
        _edge_loop(yr_hbm, src_hbm, dst_hbm, acc, src_v, dst_v, rows_v, sem,
                   s, s, CH // IBF, IBF)

    plsc.subcore_barrier()

    @pl.when(c == 0)
    def _():
        pltpu.sync_copy(acc.at[pl.ds(r0, RPT)], outl_hbm.at[pl.ds(r0, RPT)])

    @pl.when(c == 1)
    def _():
        pltpu.sync_copy(acc.at[pl.ds(r0, RPT)], outr_hbm.at[pl.ds(r0, RPT)])


def _tc_a_body(x_ref, ca_ref, cb_ref, dinv_ref, y_ref):
    di = lax.rsqrt(ca_ref[:, :1] + cb_ref[:, :1] + 1.0)
    dinv_ref[...] = di
    y_ref[...] = x_ref[...] * di


def _tc_a(x, cnt_a, cnt_b):
    return pl.pallas_call(
        _tc_a_body,
        grid=(_NBLK,),
        in_specs=[
            pl.BlockSpec((_BLK, D_IN), lambda i: (i, 0)),
            pl.BlockSpec((_BLK, 16), lambda i: (i, 0)),
            pl.BlockSpec((_BLK, 16), lambda i: (i, 0)),
        ],
        out_specs=[
            pl.BlockSpec((_BLK, 1), lambda i: (i, 0)),
            pl.BlockSpec((_BLK, D_IN), lambda i: (i, 0)),
        ],
        out_shape=[
            jax.ShapeDtypeStruct((N, 1), jnp.float32),
            jax.ShapeDtypeStruct((NP, D_IN), jnp.float32),
        ],
    )(x, cnt_a, cnt_b)


def _tc_b_body(sa_ref, sb_ref, dinv_ref, W0_ref, b0_ref, W1_ref,
               yl_ref, yr_ref):
    di = dinv_ref[...]
    a = (sa_ref[...] + sb_ref[...]) * di
    h = jnp.maximum(jnp.dot(a, W0_ref[...],
                            preferred_element_type=jnp.float32) + b0_ref[...],
                    0.0)
    y = jnp.dot(h, W1_ref[...], preferred_element_type=jnp.float32) * di
    yl_ref[...] = y[:, :HW]
    yr_ref[...] = y[:, HW:]


def _tc_b(sa, sb, dinv, W0, b0, W1):
    return pl.pallas_call(
        _tc_b_body,
        grid=(_NBLK,),
        in_specs=[
            pl.BlockSpec((_BLK, D_IN), lambda i: (i, 0)),
            pl.BlockSpec((_BLK, D_IN), lambda i: (i, 0)),
            pl.BlockSpec((_BLK, 1), lambda i: (i, 0)),
            pl.BlockSpec((D_IN, D_H), lambda i: (0, 0)),
            pl.BlockSpec((1, D_H), lambda i: (0, 0)),
            pl.BlockSpec((D_H, D_H), lambda i: (0, 0)),
        ],
        out_specs=[
            pl.BlockSpec((_BLK, HW), lambda i: (i, 0)),
            pl.BlockSpec((_BLK, HW), lambda i: (i, 0)),
        ],
        out_shape=[
            jax.ShapeDtypeStruct((NP, HW), jnp.float32),
            jax.ShapeDtypeStruct((NP, HW), jnp.float32),
        ],
    )(sa, sb, dinv, W0, b0, W1)


def _tc_c_body(sl_ref, sr_ref, dinv_ref, b_ref, W_ref, yl_ref, yr_ref):
    di = dinv_ref[...]
    h = jnp.maximum(
        jnp.concatenate([sl_ref[...], sr_ref[...]], axis=1) * di + b_ref[...],
        0.0)
    y = jnp.dot(h, W_ref[...], preferred_element_type=jnp.float32) * di
    yl_ref[...] = y[:, :HW]
    yr_ref[...] = y[:, HW:]


def _tc_c(sl, sr, dinv, b, W):
    return pl.pallas_call(
        _tc_c_body,
        grid=(_NBLK,),
        in_specs=[
            pl.BlockSpec((_BLK, HW), lambda i: (i, 0)),
            pl.BlockSpec((_BLK, HW), lambda i: (i, 0)),
            pl.BlockSpec((_BLK, 1), lambda i: (i, 0)),
            pl.BlockSpec((1, D_H), lambda i: (0, 0)),
            pl.BlockSpec((D_H, D_H), lambda i: (0, 0)),
        ],
        out_specs=[
            pl.BlockSpec((_BLK, HW), lambda i: (i, 0)),
            pl.BlockSpec((_BLK, HW), lambda i: (i, 0)),
        ],
        out_shape=[
            jax.ShapeDtypeStruct((NP, HW), jnp.float32),
            jax.ShapeDtypeStruct((NP, HW), jnp.float32),
        ],
    )(sl, sr, dinv, b, W)


def _tc_d_body(sl_ref, sr_ref, dinv_ref, b2_ref, batch_ref,
               lw1_ref, lb1_ref, lw2_ref, lb2_ref, out_ref, pooled):
    i = pl.program_id(0)
    di = dinv_ref[...]
    h = jnp.maximum(
        jnp.concatenate([sl_ref[...], sr_ref[...]], axis=1) * di + b2_ref[...],
        0.0)
    iota_b = lax.broadcasted_iota(jnp.int32, (1, B), 1)
    oh = (batch_ref[...] == iota_b).astype(jnp.float32)
    contrib = lax.dot_general(oh, h, (((0,), (0,)), ((), ())),
                              preferred_element_type=jnp.float32)

    @pl.when(i == 0)
    def _():
        pooled[...] = contrib

    @pl.when(i > 0)
    def _():
        pooled[...] += contrib

    @pl.when(i == _NBLK - 1)
    def _():
        t = jnp.maximum(
            jnp.dot(pooled[...], lw1_ref[...],
                    preferred_element_type=jnp.float32) + lb1_ref[...], 0.0)
        out_ref[...] = jnp.dot(t, lw2_ref[...],
                               preferred_element_type=jnp.float32) + lb2_ref[...]


def _tc_d(sl, sr, dinv, b2, batch2, lw1, lb1, lw2, lb2):
    return pl.pallas_call(
        _tc_d_body,
        grid=(_NBLK,),
        in_specs=[
            pl.BlockSpec((_BLK, HW), lambda i: (i, 0)),
            pl.BlockSpec((_BLK, HW), lambda i: (i, 0)),
            pl.BlockSpec((_BLK, 1), lambda i: (i, 0)),
            pl.BlockSpec((1, D_H), lambda i: (0, 0)),
            pl.BlockSpec((_BLK, 1), lambda i: (i, 0)),
            pl.BlockSpec((D_H, D_H), lambda i: (0, 0)),
            pl.BlockSpec((1, D_H), lambda i: (0, 0)),
            pl.BlockSpec((D_H, D_OUT), lambda i: (0, 0)),
            pl.BlockSpec((1, D_OUT), lambda i: (0, 0)),
        ],
        out_specs=pl.BlockSpec((B, D_OUT), lambda i: (0, 0)),
        out_shape=jax.ShapeDtypeStruct((B, D_OUT), jnp.float32),
        scratch_shapes=[pltpu.VMEM((B, D_H), jnp.float32)],
    )(sl, sr, dinv, b2, batch2, lw1, lb1, lw2, lb2)


def kernel(x, edge_index, batch, W0, b0, W1, b1, W2, b2, lw1, lb1, lw2, lb2):
    src = edge_index[0]
    dst = edge_index[1]
    pad = EP - E
    src_p = jnp.concatenate([src, jnp.zeros((pad,), jnp.int32)])
    dst_p = jnp.concatenate([dst, jnp.full((pad,), N, jnp.int32)])
    src_fs = src_p.reshape(NS, CH, CHUNK)
    dst_fs = dst_p.reshape(NS, CH, CHUNK)
    src_es = src_p.reshape(NC * NS, CH2, CHUNK)
    dst_es = dst_p.reshape(NC * NS, CH2, CHUNK)

    ones16 = jnp.ones((CHUNK, 16), jnp.float32)
    zcnt = jnp.zeros((NP, 16), jnp.float32)
    znode = jnp.zeros((NP, D_IN), jnp.float32)
    b0r = b0.reshape(1, D_H)
    b1r = b1.reshape(1, D_H)
    b2r = b2.reshape(1, D_H)
    lb1r = lb1.reshape(1, D_H)
    lb2r = lb2.reshape(1, D_OUT)
    batch2 = batch.reshape(N, 1)

    cnt_a, cnt_b = _deg_kernel(dst_es, ones16, zcnt)
    dinv, y0 = _tc_a(x, cnt_a, cnt_b)

    s0a, s0b = _scatter_es(y0, src_es, dst_es, znode)
    y1l, y1r = _tc_b(s0a, s0b, dinv, W0, b0r, W1)

    s1l, s1r = _scatter_fs(y1l, y1r, src_fs, dst_fs)
    y2l, y2r = _tc_c(s1l, s1r, dinv, b1r, W2)

    s2l, s2r = _scatter_fs(y2l, y2r, src_fs, dst_fs)
    return _tc_d(s2l, s2r, dinv, b2r, batch2, lw1, lb1r, lw2, lb2r)


# speedup vs baseline: 1.3413x; 1.3413x over previous
import functools

import jax
import jax.numpy as jnp
from jax import lax
from jax.experimental import pallas as pl
from jax.experimental.pallas import tpu as pltpu
from jax.experimental.pallas import tpu_sc as plsc

N = 10000
E = 320000
D_IN = 128
D_H = 256
D_OUT = 128
B = 128

NC = 2
NS = 16
NP = 10112
RPT = NP // NS
CHUNK = 64
CH = 316
CH2 = 158
DCHUNK = 128
DCH = 79
EP = NS * CH * CHUNK

_BLK = 400
_NBLK = N // _BLK


def _sc_mesh():
    return plsc.VectorSubcoreMesh(core_axis_name="c", subcore_axis_name="s",
                                  num_cores=NC)


@functools.partial(
    pl.kernel,
    out_type=jax.ShapeDtypeStruct((NC * NP, 16), jnp.float32),
    mesh=_sc_mesh(),
    compiler_params=pltpu.CompilerParams(use_tc_tiling_on_sc=False),
    scratch_types=[
        pltpu.VMEM_SHARED((NP, 16), jnp.float32),
        pltpu.VMEM((DCH, DCHUNK), jnp.int32),
        pltpu.VMEM((DCHUNK, 16), jnp.float32),
    ],
)
def _deg_kernel(dst_hbm, ones_hbm, zeros_hbm, out_hbm, acc, dst_v, ones_v):
    c = lax.axis_index("c")
    s = lax.axis_index("s")
    wid = c * NS + s
    r0 = s * RPT
    pltpu.sync_copy(dst_hbm.at[wid], dst_v)
    pltpu.sync_copy(ones_hbm, ones_v)
    pltpu.sync_copy(zeros_hbm.at[pl.ds(r0, RPT)], acc.at[pl.ds(r0, RPT)])
    plsc.subcore_barrier()

    @pl.loop(0, DCH)
    def _(j):
        pltpu.sync_copy(ones_v, acc.at[dst_v.at[j]], add=True)

    plsc.subcore_barrier()
    pltpu.sync_copy(acc.at[pl.ds(r0, RPT)],
                    out_hbm.at[pl.ds(c * NP + r0, RPT)])


@functools.partial(
    pl.kernel,
    out_type=jax.ShapeDtypeStruct((NC * NP, D_IN), jnp.float32),
    mesh=_sc_mesh(),
    compiler_params=pltpu.CompilerParams(use_tc_tiling_on_sc=False),
    scratch_types=[
        pltpu.VMEM_SHARED((NP, D_IN), jnp.float32),
        pltpu.VMEM((CH2, CHUNK), jnp.int32),
        pltpu.VMEM((CH2, CHUNK), jnp.int32),
        pltpu.VMEM((CHUNK, D_IN), jnp.float32),
        pltpu.SemaphoreType.DMA,
    ],
)
def _scatter_es(y_hbm, src_hbm, dst_hbm, zeros_hbm, out_hbm,
                acc, src_v, dst_v, rows_v, sem):
    c = lax.axis_index("c")
    s = lax.axis_index("s")
    wid = c * NS + s
    r0 = s * RPT
    pltpu.sync_copy(src_hbm.at[wid], src_v)
    pltpu.sync_copy(dst_hbm.at[wid], dst_v)

    @pl.when(c == 0)
    def _():
        pltpu.sync_copy(y_hbm.at[pl.ds(r0, RPT)], acc.at[pl.ds(r0, RPT)])

    @pl.when(c == 1)
    def _():
        pltpu.sync_copy(zeros_hbm.at[pl.ds(r0, RPT)], acc.at[pl.ds(r0, RPT)])

    plsc.subcore_barrier()

    @pl.loop(0, CH2)
    def _(j):
        pltpu.async_copy(y_hbm.at[src_v.at[j]], rows_v, sem).wait()
        pltpu.sync_copy(rows_v, acc.at[dst_v.at[j]], add=True)

    plsc.subcore_barrier()
    pltpu.sync_copy(acc.at[pl.ds(r0, RPT)],
                    out_hbm.at[pl.ds(c * NP + r0, RPT)])


HW = D_H // 2


@functools.partial(
    pl.kernel,
    out_type=jax.ShapeDtypeStruct((NC * NP, HW), jnp.float32),
    mesh=_sc_mesh(),
    compiler_params=pltpu.CompilerParams(use_tc_tiling_on_sc=False),
    scratch_types=[
        pltpu.VMEM_SHARED((NP, HW), jnp.float32),
        pltpu.VMEM((CH, CHUNK), jnp.int32),
        pltpu.VMEM((CH, CHUNK), jnp.int32),
        pltpu.VMEM((CHUNK, HW), jnp.float32),
        pltpu.SemaphoreType.DMA,
    ],
)
def _scatter_fs(y_hbm, src_hbm, dst_hbm, out_hbm, acc, src_v, dst_v,
                rows_v, sem):
    c = lax.axis_index("c")
    s = lax.axis_index("s")
    wid = c * NS + s
    r0 = s * RPT
    pltpu.sync_copy(src_hbm.at[wid], src_v)
    pltpu.sync_copy(dst_hbm.at[s], dst_v)
    pltpu.sync_copy(y_hbm.at[pl.ds(c * NP + r0, RPT)], acc.at[pl.ds(r0, RPT)])
    plsc.subcore_barrier()

    @pl.loop(0, CH)
    def _(j):
        pltpu.async_copy(y_hbm.at[src_v.at[j]], rows_v, sem).wait()
        pltpu.sync_copy(rows_v, acc.at[dst_v.at[j]], add=True)

    plsc.subcore_barrier()
    pltpu.sync_copy(acc.at[pl.ds(r0, RPT)],
                    out_hbm.at[pl.ds(c * NP + r0, RPT)])


def _tc_a_body(x_ref, ca_ref, cb_ref, dinv_ref, y_ref):
    di = lax.rsqrt(ca_ref[:, :1] + cb_ref[:, :1] + 1.0)
    dinv_ref[...] = di
    y_ref[...] = x_ref[...] * di


def _tc_a(x, cnt_a, cnt_b):
    return pl.pallas_call(
        _tc_a_body,
        grid=(_NBLK,),
        in_specs=[
            pl.BlockSpec((_BLK, D_IN), lambda i: (i, 0)),
            pl.BlockSpec((_BLK, 16), lambda i: (i, 0)),
            pl.BlockSpec((_BLK, 16), lambda i: (i, 0)),
        ],
        out_specs=[
            pl.BlockSpec((_BLK, 1), lambda i: (i, 0)),
            pl.BlockSpec((_BLK, D_IN), lambda i: (i, 0)),
        ],
        out_shape=[
            jax.ShapeDtypeStruct((N, 1), jnp.float32),
            jax.ShapeDtypeStruct((N, D_IN), jnp.float32),
        ],
    )(x, cnt_a, cnt_b)


def _tc_b_body(sa_ref, sb_ref, dinv_ref, W0_ref, b0_ref, W1_ref,
               yl_ref, yr_ref):
    di = dinv_ref[...]
    a = (sa_ref[...] + sb_ref[...]) * di
    h = jnp.maximum(jnp.dot(a, W0_ref[...],
                            preferred_element_type=jnp.float32) + b0_ref[...],
                    0.0)
    y = jnp.dot(h, W1_ref[...], preferred_element_type=jnp.float32) * di
    yl_ref[...] = y[:, :HW]
    yr_ref[...] = y[:, HW:]


def _tc_b(sa, sb, dinv, W0, b0, W1):
    return pl.pallas_call(
        _tc_b_body,
        grid=(_NBLK,),
        in_specs=[
            pl.BlockSpec((_BLK, D_IN), lambda i: (i, 0)),
            pl.BlockSpec((_BLK, D_IN), lambda i: (i, 0)),
            pl.BlockSpec((_BLK, 1), lambda i: (i, 0)),
            pl.BlockSpec((D_IN, D_H), lambda i: (0, 0)),
            pl.BlockSpec((1, D_H), lambda i: (0, 0)),
            pl.BlockSpec((D_H, D_H), lambda i: (0, 0)),
        ],
        out_specs=[
            pl.BlockSpec((_BLK, HW), lambda i: (i, 0)),
            pl.BlockSpec((_BLK, HW), lambda i: (i, 0)),
        ],
        out_shape=[
            jax.ShapeDtypeStruct((N, HW), jnp.float32),
            jax.ShapeDtypeStruct((N, HW), jnp.float32),
        ],
    )(sa, sb, dinv, W0, b0, W1)


def _tc_c_body(sl_ref, sr_ref, dinv_ref, b_ref, W_ref, yl_ref, yr_ref):
    di = dinv_ref[...]
    h = jnp.maximum(
        jnp.concatenate([sl_ref[...], sr_ref[...]], axis=1) * di + b_ref[...],
        0.0)
    y = jnp.dot(h, W_ref[...], preferred_element_type=jnp.float32) * di
    yl_ref[...] = y[:, :HW]
    yr_ref[...] = y[:, HW:]


def _tc_c(sl, sr, dinv, b, W):
    return pl.pallas_call(
        _tc_c_body,
        grid=(_NBLK,),
        in_specs=[
            pl.BlockSpec((_BLK, HW), lambda i: (i, 0)),
            pl.BlockSpec((_BLK, HW), lambda i: (i, 0)),
            pl.BlockSpec((_BLK, 1), lambda i: (i, 0)),
            pl.BlockSpec((1, D_H), lambda i: (0, 0)),
            pl.BlockSpec((D_H, D_H), lambda i: (0, 0)),
        ],
        out_specs=[
            pl.BlockSpec((_BLK, HW), lambda i: (i, 0)),
            pl.BlockSpec((_BLK, HW), lambda i: (i, 0)),
        ],
        out_shape=[
            jax.ShapeDtypeStruct((N, HW), jnp.float32),
            jax.ShapeDtypeStruct((N, HW), jnp.float32),
        ],
    )(sl, sr, dinv, b, W)


def _tc_d_body(sl_ref, sr_ref, dinv_ref, b2_ref, batch_ref,
               lw1_ref, lb1_ref, lw2_ref, lb2_ref, out_ref, pooled):
    i = pl.program_id(0)
    di = dinv_ref[...]
    h = jnp.maximum(
        jnp.concatenate([sl_ref[...], sr_ref[...]], axis=1) * di + b2_ref[...],
        0.0)
    iota_b = lax.broadcasted_iota(jnp.int32, (1, B), 1)
    oh = (batch_ref[...] == iota_b).astype(jnp.float32)
    contrib = lax.dot_general(oh, h, (((0,), (0,)), ((), ())),
                              preferred_element_type=jnp.float32)

    @pl.when(i == 0)
    def _():
        pooled[...] = contrib

    @pl.when(i > 0)
    def _():
        pooled[...] += contrib

    @pl.when(i == _NBLK - 1)
    def _():
        t = jnp.maximum(
            jnp.dot(pooled[...], lw1_ref[...],
                    preferred_element_type=jnp.float32) + lb1_ref[...], 0.0)
        out_ref[...] = jnp.dot(t, lw2_ref[...],
                               preferred_element_type=jnp.float32) + lb2_ref[...]


def _tc_d(sl, sr, dinv, b2, batch2, lw1, lb1, lw2, lb2):
    return pl.pallas_call(
        _tc_d_body,
        grid=(_NBLK,),
        in_specs=[
            pl.BlockSpec((_BLK, HW), lambda i: (i, 0)),
            pl.BlockSpec((_BLK, HW), lambda i: (i, 0)),
            pl.BlockSpec((_BLK, 1), lambda i: (i, 0)),
            pl.BlockSpec((1, D_H), lambda i: (0, 0)),
            pl.BlockSpec((_BLK, 1), lambda i: (i, 0)),
            pl.BlockSpec((D_H, D_H), lambda i: (0, 0)),
            pl.BlockSpec((1, D_H), lambda i: (0, 0)),
            pl.BlockSpec((D_H, D_OUT), lambda i: (0, 0)),
            pl.BlockSpec((1, D_OUT), lambda i: (0, 0)),
        ],
        out_specs=pl.BlockSpec((B, D_OUT), lambda i: (0, 0)),
        out_shape=jax.ShapeDtypeStruct((B, D_OUT), jnp.float32),
        scratch_shapes=[pltpu.VMEM((B, D_H), jnp.float32)],
    )(sl, sr, dinv, b2, batch2, lw1, lb1, lw2, lb2)


def kernel(x, edge_index, batch, W0, b0, W1, b1, W2, b2, lw1, lb1, lw2, lb2):
    src = edge_index[0]
    dst = edge_index[1]
    pad = EP - E
    src_p = jnp.concatenate([src, jnp.zeros((pad,), jnp.int32)])
    dst_p = jnp.concatenate([dst, jnp.full((pad,), N, jnp.int32)])
    src_fs = src_p.reshape(NS, CH, CHUNK)
    dst_fs = dst_p.reshape(NS, CH, CHUNK)
    src_fs_off = jnp.concatenate([src_fs, src_fs + NP]).reshape(
        NC * NS, CH, CHUNK)
    src_es = src_p.reshape(NC * NS, CH2, CHUNK)
    dst_es = dst_p.reshape(NC * NS, CH2, CHUNK)
    dst_dg = dst_p.reshape(NC * NS, DCH, DCHUNK)

    ones16 = jnp.ones((DCHUNK, 16), jnp.float32)
    zcnt = jnp.zeros((NP, 16), jnp.float32)
    zrow = jnp.zeros((NP - N, D_IN), jnp.float32)
    zrow_h = jnp.zeros((NP - N, HW), jnp.float32)
    znode = jnp.zeros((NP, D_IN), jnp.float32)
    b0r = b0.reshape(1, D_H)
    b1r = b1.reshape(1, D_H)
    b2r = b2.reshape(1, D_H)
    lb1r = lb1.reshape(1, D_H)
    lb2r = lb2.reshape(1, D_OUT)
    batch2 = batch.reshape(N, 1)

    cnt = _deg_kernel(dst_dg, ones16, zcnt)
    dinv, y0 = _tc_a(x, cnt[:N], cnt[NP:NP + N])

    y0f = jnp.concatenate([y0, zrow])
    s0f = _scatter_es(y0f, src_es, dst_es, znode)
    y1l, y1r = _tc_b(s0f[:N], s0f[NP:NP + N], dinv, W0, b0r, W1)

    y1f = jnp.concatenate([y1l, zrow_h, y1r, zrow_h])
    s1f = _scatter_fs(y1f, src_fs_off, dst_fs)
    y2l, y2r = _tc_c(s1f[:N], s1f[NP:NP + N], dinv, b1r, W2)

    y2f = jnp.concatenate([y2l, zrow_h, y2r, zrow_h])
    s2f = _scatter_fs(y2f, src_fs_off, dst_fs)
    return _tc_d(s2f[:N], s2f[NP:NP + N], dinv, b2r, batch2,
                 lw1, lb1r, lw2, lb2r)
